# Initial kernel scaffold; baseline (speedup 1.0000x reference)
import jax
import jax.numpy as jnp
from jax.experimental import pallas as pl


def kernel(x, edge_index, Wx, bx, Wgat, al, ar, bgat, Wi, bi, Wf, bf, Wo, bo, Wc, bc, Wout, bout):
    raise NotImplementedError("write your pallas kernel here")


# speedup vs baseline: 19.6191x; 19.6191x over previous
import dataclasses
import functools

import jax
import jax.numpy as jnp
from jax import lax
from jax.experimental import pallas as pl
from jax.experimental.pallas import tpu as pltpu
from jax.experimental.pallas import tpu_sc as plsc

N = 10000
E = 320000
H = 128
DEPTH = 3

NB = 400
GRID = N // NB

NC = 2
NS = 16
NW = NC * NS
EPW = E // NW
K = 80
CH = EPW // K
ZR = 80
NCHUNK = N // ZR


def _affine_body(x_ref, w_ref, b_ref, o_ref, *, act):
    y = jnp.dot(x_ref[...], w_ref[...], preferred_element_type=jnp.float32)
    y = y + b_ref[...]
    if act == "relu":
        y = jnp.maximum(y, 0.0)
    o_ref[...] = y


def _tc_affine(x, w, b, act):
    return pl.pallas_call(
        functools.partial(_affine_body, act=act),
        grid=(GRID,),
        in_specs=[
            pl.BlockSpec((NB, H), lambda i: (i, 0)),
            pl.BlockSpec((H, H), lambda i: (0, 0)),
            pl.BlockSpec((1, H), lambda i: (0, 0)),
        ],
        out_specs=pl.BlockSpec((NB, H), lambda i: (i, 0)),
        out_shape=jax.ShapeDtypeStruct((N, H), jnp.float32),
    )(x, w, b.reshape(1, H))


def _pre_body(h_ref, w_ref, al_ref, ar_ref, z_ref, el_ref, er_ref, c_ref,
              m_ref):
    i = pl.program_id(0)
    z = jnp.dot(h_ref[...], w_ref[...], preferred_element_type=jnp.float32)
    z_ref[...] = z
    el = jnp.sum(z * al_ref[...], axis=1, keepdims=True)
    er = jnp.sum(z * ar_ref[...], axis=1, keepdims=True)
    el_ref[...] = el
    er_ref[...] = er
    bl = jnp.max(el)
    br = jnp.max(er)

    @pl.when(i == 0)
    def _():
        m_ref[0] = bl
        m_ref[1] = br

    @pl.when(i > 0)
    def _():
        m_ref[0] = jnp.maximum(m_ref[0], bl)
        m_ref[1] = jnp.maximum(m_ref[1], br)

    @pl.when(i == GRID - 1)
    def _():
        c_ref[...] = jnp.full((1, H), jnp.maximum(m_ref[0] + m_ref[1], 0.0),
                              dtype=jnp.float32)


def _tc_pre(h, w, al, ar):
    return pl.pallas_call(
        _pre_body,
        grid=(GRID,),
        in_specs=[
            pl.BlockSpec((NB, H), lambda i: (i, 0)),
            pl.BlockSpec((H, H), lambda i: (0, 0)),
            pl.BlockSpec((1, H), lambda i: (0, 0)),
            pl.BlockSpec((1, H), lambda i: (0, 0)),
        ],
        out_specs=[
            pl.BlockSpec((NB, H), lambda i: (i, 0)),
            pl.BlockSpec((NB, 1), lambda i: (i, 0)),
            pl.BlockSpec((NB, 1), lambda i: (i, 0)),
            pl.BlockSpec((1, H), lambda i: (0, 0)),
        ],
        out_shape=[
            jax.ShapeDtypeStruct((N, H), jnp.float32),
            jax.ShapeDtypeStruct((N, 1), jnp.float32),
            jax.ShapeDtypeStruct((N, 1), jnp.float32),
            jax.ShapeDtypeStruct((1, H), jnp.float32),
        ],
        scratch_shapes=[pltpu.SMEM((2,), jnp.float32)],
    )(h, w, al.reshape(1, H), ar.reshape(1, H))


def _denred_body(denp_ref, den_ref):
    d = jnp.sum(denp_ref[...], axis=0)
    den_ref[...] = d.reshape(N, 1)


def _tc_denred(denp):
    return pl.pallas_call(
        _denred_body,
        grid=(1,),
        in_specs=[pl.BlockSpec((NW, N), lambda i: (0, 0))],
        out_specs=pl.BlockSpec((N, 1), lambda i: (0, 0)),
        out_shape=jax.ShapeDtypeStruct((N, 1), jnp.float32),
    )(denp)


def _post_body(acc_ref, den_ref, b_ref, h_ref):
    acc = acc_ref[0] + acc_ref[1]
    h_ref[...] = jnp.tanh(acc / (den_ref[...] + 1e-16) + b_ref[...])


def _tc_post(accp, den, b):
    return pl.pallas_call(
        _post_body,
        grid=(GRID,),
        in_specs=[
            pl.BlockSpec((NC, NB, H), lambda i: (0, i, 0)),
            pl.BlockSpec((NB, 1), lambda i: (i, 0)),
            pl.BlockSpec((1, H), lambda i: (0, 0)),
        ],
        out_specs=pl.BlockSpec((NB, H), lambda i: (i, 0)),
        out_shape=jax.ShapeDtypeStruct((N, H), jnp.float32),
    )(accp, den, b.reshape(1, H))


def _lstm_body(h_ref, mu_ref, c_ref, wi_ref, wf_ref, wo_ref, wc_ref,
               bi_ref, bf_ref, bo_ref, bc_ref, co_ref, muo_ref):
    a = h_ref[...]
    m = mu_ref[...]

    def gate(w_ref, b_ref):
        y = jnp.dot(a, w_ref[0:H, :], preferred_element_type=jnp.float32)
        y = y + jnp.dot(m, w_ref[H:2 * H, :],
                        preferred_element_type=jnp.float32)
        return y + b_ref[...]

    ig = jax.nn.sigmoid(gate(wi_ref, bi_ref))
    fg = jax.nn.sigmoid(gate(wf_ref, bf_ref))
    og = jax.nn.sigmoid(gate(wo_ref, bo_ref))
    ct = jnp.tanh(gate(wc_ref, bc_ref))
    c = fg * c_ref[...] + ig * ct
    co_ref[...] = c
    muo_ref[...] = og * jnp.tanh(c)


def _tc_lstm(h, mu, c, wi, wf, wo, wc, bi, bf, bo, bc):
    wspec = pl.BlockSpec((2 * H, H), lambda i: (0, 0))
    bspec = pl.BlockSpec((1, H), lambda i: (0, 0))
    nspec = pl.BlockSpec((NB, H), lambda i: (i, 0))
    return pl.pallas_call(
        _lstm_body,
        grid=(GRID,),
        in_specs=[nspec, nspec, nspec, wspec, wspec, wspec, wspec,
                  bspec, bspec, bspec, bspec],
        out_specs=[nspec, nspec],
        out_shape=[jax.ShapeDtypeStruct((N, H), jnp.float32),
                   jax.ShapeDtypeStruct((N, H), jnp.float32)],
    )(h, mu, c, wi, wf, wo, wc, bi.reshape(1, H), bf.reshape(1, H),
      bo.reshape(1, H), bc.reshape(1, H))


def _sc_edge(z, el, er, src, dst, cshift):
    mesh = plsc.VectorSubcoreMesh(core_axis_name="c", subcore_axis_name="s")
    cp = pltpu.CompilerParams()
    if "needs_layout_passes" in pltpu.CompilerParams.__dataclass_fields__:
        cp = dataclasses.replace(cp, needs_layout_passes=False)

    @functools.partial(
        pl.kernel,
        compiler_params=cp,
        out_type=(
            jax.ShapeDtypeStruct((NC, N, H), jnp.float32),
            jax.ShapeDtypeStruct((NW, N), jnp.float32),
        ),
        mesh=mesh,
        scratch_types=[
            pltpu.VMEM((N,), jnp.float32),
            pltpu.VMEM((N,), jnp.float32),
            pltpu.VMEM((N,), jnp.float32),
            pltpu.VMEM((K,), jnp.int32),
            pltpu.VMEM((K,), jnp.int32),
            pltpu.VMEM((K,), jnp.float32),
            pltpu.VMEM((K, H), jnp.float32),
            pltpu.VMEM((16,), jnp.float32),
            pltpu.VMEM_SHARED((N, H), jnp.float32),
            pltpu.SemaphoreType.DMA,
        ],
    )
    def k(z_hbm, el_hbm, er_hbm, src_hbm, dst_hbm, c_hbm, accout, denout,
          el_v, er_v, den_v, src_v, dst_v, w_v, rows_v, c_v,
          acc_sh, sem):
        core = lax.axis_index("c")
        sub = lax.axis_index("s")
        wid = core * NS + sub

        pltpu.sync_copy(el_hbm, el_v)
        pltpu.sync_copy(er_hbm, er_v)
        pltpu.sync_copy(c_hbm, c_v)
        cvec = c_v[...]
        zero16 = jnp.zeros((16,), jnp.float32)

        @pl.loop(0, ZR)
        def _(i):
            for cc in range(H // 16):
                rows_v[i, pl.ds(cc * 16, 16)] = zero16

        @pl.loop(0, N, step=16)
        def _(i):
            den_v[pl.ds(i, 16)] = zero16

        for t in range((NCHUNK + NS - 1) // NS):
            cid = sub + NS * t

            @pl.when(cid < NCHUNK)
            def _():
                pltpu.sync_copy(rows_v, acc_sh.at[pl.ds(cid * ZR, ZR)])
        plsc.subcore_barrier()

        @pl.loop(0, CH)
        def _(g):
            base = wid * EPW + g * K
            pltpu.sync_copy(src_hbm.at[pl.ds(base, K)], src_v)
            pltpu.sync_copy(dst_hbm.at[pl.ds(base, K)], dst_v)
            pltpu.async_copy(z_hbm.at[src_v], rows_v, sem).wait()
            for j in range(K // 16):
                sidx = src_v[pl.ds(j * 16, 16)]
                didx = dst_v[pl.ds(j * 16, 16)]
                t = plsc.load_gather(el_v, [sidx]) \
                    + plsc.load_gather(er_v, [didx])
                e = jnp.where(t >= 0.0, t, 0.2 * t)
                w = jnp.exp(e - cvec)
                w_v[pl.ds(j * 16, 16)] = w
                plsc.addupdate_scatter(den_v, [didx], w)

            @pl.loop(0, K)
            def _(r):
                wsplat = plsc.load_gather(w_v, [lax.broadcast(r, (16,))])
                for cc in range(H // 16):
                    sl = pl.ds(cc * 16, 16)
                    rows_v[r, sl] = rows_v[r, sl] * wsplat

            pltpu.sync_copy(rows_v, acc_sh.at[dst_v], add=True)

        plsc.subcore_barrier()
        for t in range((NCHUNK + NS - 1) // NS):
            cid = sub + NS * t

            @pl.when(cid < NCHUNK)
            def _():
                rows = pl.ds(cid * ZR, ZR)
                pltpu.sync_copy(acc_sh.at[rows], accout.at[core, rows])
        pltpu.sync_copy(den_v, denout.at[wid])

    return k(z, el, er, src, dst, cshift)


def kernel(x, edge_index, Wx, bx, Wgat, al, ar, bgat, Wi, bi, Wf, bf,
           Wo, bo, Wc, bc, Wout, bout):
    src = edge_index[0]
    dst = edge_index[1]

    h0 = _tc_affine(x, Wx, bx, act="none")
    h = h0
    collector = []
    for i in range(DEPTH):
        z, el, er, cmat = _tc_pre(h, Wgat[i], al[i], ar[i])
        cshift = cmat[0, :16]
        accp, denp = _sc_edge(z, el.reshape(N), er.reshape(N), src, dst,
                              cshift)
        h = _tc_post(accp, _tc_denred(denp), bgat[i])
        collector.append(h)

    mu = h0
    c = jnp.zeros_like(mu)
    for i in range(DEPTH):
        c, mu = _tc_lstm(collector[i], mu, c, Wi[i], Wf[i], Wo[i], Wc[i],
                         bi[i], bf[i], bo[i], bc[i])

    return _tc_affine(mu, Wout, bout, act="relu")


# speedup vs baseline: 32.4974x; 1.6564x over previous
import dataclasses
import functools

import jax
import jax.numpy as jnp
from jax import lax
from jax.experimental import pallas as pl
from jax.experimental.pallas import tpu as pltpu
from jax.experimental.pallas import tpu_sc as plsc

N = 10000
E = 320000
H = 128
DEPTH = 3

NB = 400
GRID = N // NB

NC = 2
NS = 16
NW = NC * NS
K = 128
NG = E // K
GPW = (NG + NW - 1) // NW
ZR = 80
NCHUNK = N // ZR


def _affine_body(x_ref, w_ref, b_ref, o_ref, *, act):
    y = jnp.dot(x_ref[...], w_ref[...], preferred_element_type=jnp.float32)
    y = y + b_ref[...]
    if act == "relu":
        y = jnp.maximum(y, 0.0)
    o_ref[...] = y


def _tc_affine(x, w, b, act):
    return pl.pallas_call(
        functools.partial(_affine_body, act=act),
        grid=(GRID,),
        in_specs=[
            pl.BlockSpec((NB, H), lambda i: (i, 0)),
            pl.BlockSpec((H, H), lambda i: (0, 0)),
            pl.BlockSpec((1, H), lambda i: (0, 0)),
        ],
        out_specs=pl.BlockSpec((NB, H), lambda i: (i, 0)),
        out_shape=jax.ShapeDtypeStruct((N, H), jnp.float32),
    )(x, w, b.reshape(1, H))


def _pre_body(h_ref, w_ref, al_ref, ar_ref, z_ref, el_ref, er_ref, c_ref,
              m_ref):
    i = pl.program_id(0)
    z = jnp.dot(h_ref[...], w_ref[...], preferred_element_type=jnp.float32)
    z_ref[...] = z
    el = jnp.sum(z * al_ref[...], axis=1, keepdims=True)
    er = jnp.sum(z * ar_ref[...], axis=1, keepdims=True)
    el_ref[...] = el
    er_ref[...] = er
    bl = jnp.max(el)
    br = jnp.max(er)

    @pl.when(i == 0)
    def _():
        m_ref[0] = bl
        m_ref[1] = br

    @pl.when(i > 0)
    def _():
        m_ref[0] = jnp.maximum(m_ref[0], bl)
        m_ref[1] = jnp.maximum(m_ref[1], br)

    @pl.when(i == GRID - 1)
    def _():
        c_ref[...] = jnp.full((1, H), jnp.maximum(m_ref[0] + m_ref[1], 0.0),
                              dtype=jnp.float32)


def _tc_pre(h, w, al, ar):
    return pl.pallas_call(
        _pre_body,
        grid=(GRID,),
        in_specs=[
            pl.BlockSpec((NB, H), lambda i: (i, 0)),
            pl.BlockSpec((H, H), lambda i: (0, 0)),
            pl.BlockSpec((1, H), lambda i: (0, 0)),
            pl.BlockSpec((1, H), lambda i: (0, 0)),
        ],
        out_specs=[
            pl.BlockSpec((NB, H), lambda i: (i, 0)),
            pl.BlockSpec((NB, 1), lambda i: (i, 0)),
            pl.BlockSpec((NB, 1), lambda i: (i, 0)),
            pl.BlockSpec((1, H), lambda i: (0, 0)),
        ],
        out_shape=[
            jax.ShapeDtypeStruct((N, H), jnp.float32),
            jax.ShapeDtypeStruct((N, 1), jnp.float32),
            jax.ShapeDtypeStruct((N, 1), jnp.float32),
            jax.ShapeDtypeStruct((1, H), jnp.float32),
        ],
        scratch_shapes=[pltpu.SMEM((2,), jnp.float32)],
    )(h, w, al.reshape(1, H), ar.reshape(1, H))


def _denred_body(denp_ref, den_ref):
    d = jnp.sum(denp_ref[...], axis=0)
    den_ref[...] = d.reshape(N, 1)


def _tc_denred(denp):
    return pl.pallas_call(
        _denred_body,
        grid=(1,),
        in_specs=[pl.BlockSpec((NW, N), lambda i: (0, 0))],
        out_specs=pl.BlockSpec((N, 1), lambda i: (0, 0)),
        out_shape=jax.ShapeDtypeStruct((N, 1), jnp.float32),
    )(denp)


def _post_body(acc_ref, den_ref, b_ref, h_ref):
    acc = acc_ref[0] + acc_ref[1]
    h_ref[...] = jnp.tanh(acc / (den_ref[...] + 1e-16) + b_ref[...])


def _tc_post(accp, den, b):
    return pl.pallas_call(
        _post_body,
        grid=(GRID,),
        in_specs=[
            pl.BlockSpec((NC, NB, H), lambda i: (0, i, 0)),
            pl.BlockSpec((NB, 1), lambda i: (i, 0)),
            pl.BlockSpec((1, H), lambda i: (0, 0)),
        ],
        out_specs=pl.BlockSpec((NB, H), lambda i: (i, 0)),
        out_shape=jax.ShapeDtypeStruct((N, H), jnp.float32),
    )(accp, den, b.reshape(1, H))


def _lstm_body(h_ref, mu_ref, c_ref, wi_ref, wf_ref, wo_ref, wc_ref,
               bi_ref, bf_ref, bo_ref, bc_ref, co_ref, muo_ref):
    a = h_ref[...]
    m = mu_ref[...]

    def gate(w_ref, b_ref):
        y = jnp.dot(a, w_ref[0:H, :], preferred_element_type=jnp.float32)
        y = y + jnp.dot(m, w_ref[H:2 * H, :],
                        preferred_element_type=jnp.float32)
        return y + b_ref[...]

    ig = jax.nn.sigmoid(gate(wi_ref, bi_ref))
    fg = jax.nn.sigmoid(gate(wf_ref, bf_ref))
    og = jax.nn.sigmoid(gate(wo_ref, bo_ref))
    ct = jnp.tanh(gate(wc_ref, bc_ref))
    c = fg * c_ref[...] + ig * ct
    co_ref[...] = c
    muo_ref[...] = og * jnp.tanh(c)


def _tc_lstm(h, mu, c, wi, wf, wo, wc, bi, bf, bo, bc):
    wspec = pl.BlockSpec((2 * H, H), lambda i: (0, 0))
    bspec = pl.BlockSpec((1, H), lambda i: (0, 0))
    nspec = pl.BlockSpec((NB, H), lambda i: (i, 0))
    return pl.pallas_call(
        _lstm_body,
        grid=(GRID,),
        in_specs=[nspec, nspec, nspec, wspec, wspec, wspec, wspec,
                  bspec, bspec, bspec, bspec],
        out_specs=[nspec, nspec],
        out_shape=[jax.ShapeDtypeStruct((N, H), jnp.float32),
                   jax.ShapeDtypeStruct((N, H), jnp.float32)],
    )(h, mu, c, wi, wf, wo, wc, bi.reshape(1, H), bf.reshape(1, H),
      bo.reshape(1, H), bc.reshape(1, H))


def _sc_edge(z, el, er, src, dst, cshift):
    mesh = plsc.VectorSubcoreMesh(core_axis_name="c", subcore_axis_name="s")
    cp = pltpu.CompilerParams()
    if "needs_layout_passes" in pltpu.CompilerParams.__dataclass_fields__:
        cp = dataclasses.replace(cp, needs_layout_passes=False)

    @functools.partial(
        pl.kernel,
        compiler_params=cp,
        out_type=(
            jax.ShapeDtypeStruct((NC, N, H), jnp.float32),
            jax.ShapeDtypeStruct((NW, N), jnp.float32),
        ),
        mesh=mesh,
        scratch_types=[
            pltpu.VMEM((N,), jnp.float32),
            pltpu.VMEM((2, K), jnp.int32),
            pltpu.VMEM((2, K), jnp.int32),
            pltpu.VMEM((2, K), jnp.float32),
            pltpu.VMEM((2, K), jnp.float32),
            pltpu.VMEM((2, K, H), jnp.float32),
            pltpu.VMEM((K,), jnp.float32),
            pltpu.VMEM((16,), jnp.float32),
            pltpu.VMEM_SHARED((N, H), jnp.float32),
            pltpu.SemaphoreType.DMA,
            pltpu.SemaphoreType.DMA,
            pltpu.SemaphoreType.DMA,
            pltpu.SemaphoreType.DMA,
        ],
    )
    def k(z_hbm, el_hbm, er_hbm, src_hbm, dst_hbm, c_hbm, accout, denout,
          den_v, src_v, dst_v, el_v, er_v, rows_v, w_v, c_v,
          acc_sh, semi0, semi1, semg0, semg1):
        core = lax.axis_index("c")
        sub = lax.axis_index("s")
        wid = core * NS + sub
        semi = (semi0, semi1)
        semg = (semg0, semg1)

        pltpu.sync_copy(c_hbm, c_v)
        cvec = c_v[...]
        zero16 = jnp.zeros((16,), jnp.float32)

        @pl.loop(0, ZR)
        def _(i):
            for cc in range(H // 16):
                rows_v[0, i, pl.ds(cc * 16, 16)] = zero16

        @pl.loop(0, N, step=16)
        def _(i):
            den_v[pl.ds(i, 16)] = zero16

        for t in range((NCHUNK + NS - 1) // NS):
            cid = sub + NS * t

            @pl.when(cid < NCHUNK)
            def _():
                pltpu.sync_copy(rows_v.at[0, pl.ds(0, ZR)],
                                acc_sh.at[pl.ds(cid * ZR, ZR)])
        plsc.subcore_barrier()

        def idx_copies(g, b):
            cid = g * NW + wid
            base = cid * K
            return (
                pltpu.make_async_copy(src_hbm.at[pl.ds(base, K)],
                                      src_v.at[b], semi[b]),
                pltpu.make_async_copy(dst_hbm.at[pl.ds(base, K)],
                                      dst_v.at[b], semi[b]),
            )

        def gather_copies(b):
            return (
                pltpu.make_async_copy(el_hbm.at[src_v.at[b]], el_v.at[b],
                                      semg[b]),
                pltpu.make_async_copy(er_hbm.at[dst_v.at[b]], er_v.at[b],
                                      semg[b]),
                pltpu.make_async_copy(z_hbm.at[src_v.at[b]], rows_v.at[b],
                                      semg[b]),
            )

        def active(g):
            return g * NW + wid < NG

        def issue_idx(g, b):
            @pl.when(active(g))
            def _():
                for d in idx_copies(g, b):
                    d.start()

        def wait_idx(g, b):
            @pl.when(active(g))
            def _():
                for d in idx_copies(g, b):
                    d.wait()

        def issue_gathers(g, b):
            @pl.when(active(g))
            def _():
                for d in gather_copies(b):
                    d.start()

        def wait_gathers(g, b):
            @pl.when(active(g))
            def _():
                for d in gather_copies(b):
                    d.wait()

        def compute_scatter(g, b):
            @pl.when(active(g))
            def _():
                for j in range(K // 16):
                    sl = pl.ds(j * 16, 16)
                    didx = dst_v[b, sl]
                    t = el_v[b, sl] + er_v[b, sl]
                    e = jnp.where(t >= 0.0, t, 0.2 * t)
                    w = jnp.exp(e - cvec)
                    w_v[sl] = w
                    plsc.addupdate_scatter(den_v, [didx], w)

                @pl.loop(0, K)
                def _(r):
                    wsplat = plsc.load_gather(w_v, [lax.broadcast(r, (16,))])
                    for cc in range(H // 16):
                        sl = pl.ds(cc * 16, 16)
                        rows_v[b, r, sl] = rows_v[b, r, sl] * wsplat

                pltpu.sync_copy(rows_v.at[b], acc_sh.at[dst_v.at[b]],
                                add=True)

        issue_idx(0, 0)
        issue_idx(1, 1)
        wait_idx(0, 0)
        issue_gathers(0, 0)

        @pl.loop(0, (GPW + 1) // 2)
        def _(h):
            for par in range(2):
                g = 2 * h + par
                b = par
                nb = 1 - par
                wait_idx(g + 1, nb)
                issue_gathers(g + 1, nb)
                wait_gathers(g, b)
                compute_scatter(g, b)
                issue_idx(g + 2, b)

        plsc.subcore_barrier()
        for t in range((NCHUNK + NS - 1) // NS):
            cid = sub + NS * t

            @pl.when(cid < NCHUNK)
            def _():
                rows = pl.ds(cid * ZR, ZR)
                pltpu.sync_copy(acc_sh.at[rows], accout.at[core, rows])
        pltpu.sync_copy(den_v, denout.at[wid])

    return k(z, el, er, src, dst, cshift)


def kernel(x, edge_index, Wx, bx, Wgat, al, ar, bgat, Wi, bi, Wf, bf,
           Wo, bo, Wc, bc, Wout, bout):
    src = edge_index[0]
    dst = edge_index[1]

    h0 = _tc_affine(x, Wx, bx, act="none")
    h = h0
    collector = []
    for i in range(DEPTH):
        z, el, er, cmat = _tc_pre(h, Wgat[i], al[i], ar[i])
        cshift = cmat[0, :16]
        accp, denp = _sc_edge(z, el.reshape(N), er.reshape(N), src, dst,
                              cshift)
        h = _tc_post(accp, _tc_denred(denp), bgat[i])
        collector.append(h)

    mu = h0
    c = jnp.zeros_like(mu)
    for i in range(DEPTH):
        c, mu = _tc_lstm(collector[i], mu, c, Wi[i], Wf[i], Wo[i], Wc[i],
                         bi[i], bf[i], bo[i], bc[i])

    return _tc_affine(mu, Wout, bout, act="relu")


# speedup vs baseline: 41.1539x; 1.2664x over previous
import dataclasses
import functools

import jax
import jax.numpy as jnp
from jax import lax
from jax.experimental import pallas as pl
from jax.experimental.pallas import tpu as pltpu
from jax.experimental.pallas import tpu_sc as plsc

N = 10000
E = 320000
H = 128
DEPTH = 3

NB = 400
GRID = N // NB

NC = 2
NS = 16
NW = NC * NS
K = 128
NG = E // K
GPW = (NG + NW - 1) // NW
ZR = 80
NCHUNK = N // ZR


def _affine_body(x_ref, w_ref, b_ref, o_ref, *, act):
    y = jnp.dot(x_ref[...], w_ref[...], preferred_element_type=jnp.float32)
    y = y + b_ref[...]
    if act == "relu":
        y = jnp.maximum(y, 0.0)
    o_ref[...] = y


def _tc_affine(x, w, b, act):
    return pl.pallas_call(
        functools.partial(_affine_body, act=act),
        grid=(GRID,),
        in_specs=[
            pl.BlockSpec((NB, H), lambda i: (i, 0)),
            pl.BlockSpec((H, H), lambda i: (0, 0)),
            pl.BlockSpec((1, H), lambda i: (0, 0)),
        ],
        out_specs=pl.BlockSpec((NB, H), lambda i: (i, 0)),
        out_shape=jax.ShapeDtypeStruct((N, H), jnp.float32),
    )(x, w, b.reshape(1, H))


def _pre_body(h_ref, w_ref, al_ref, ar_ref, z_ref, el_ref, er_ref, c_ref,
              m_ref, *, wx_ref=None, bx_ref=None, h_out_ref=None,
              accp_ref=None, den_ref=None, bg_ref=None):
    i = pl.program_id(0)
    if wx_ref is not None:
        h = jnp.dot(h_ref[...], wx_ref[...],
                    preferred_element_type=jnp.float32) + bx_ref[...]
        h_out_ref[...] = h
    elif accp_ref is not None:
        acc = accp_ref[0] + accp_ref[1]
        h = jnp.tanh(acc / (den_ref[...] + 1e-16) + bg_ref[...])
        h_out_ref[...] = h
    else:
        h = h_ref[...]
    z = jnp.dot(h, w_ref[...], preferred_element_type=jnp.float32)
    z_ref[...] = z
    el = jnp.sum(z * al_ref[...], axis=1, keepdims=True)
    er = jnp.sum(z * ar_ref[...], axis=1, keepdims=True)
    el_ref[...] = el
    er_ref[...] = er
    bl = jnp.max(el)
    br = jnp.max(er)

    @pl.when(i == 0)
    def _():
        m_ref[0] = bl
        m_ref[1] = br

    @pl.when(i > 0)
    def _():
        m_ref[0] = jnp.maximum(m_ref[0], bl)
        m_ref[1] = jnp.maximum(m_ref[1], br)

    @pl.when(i == GRID - 1)
    def _():
        c_ref[...] = jnp.full((1, H), jnp.maximum(m_ref[0] + m_ref[1], 0.0),
                              dtype=jnp.float32)


_PRE_OUT_SPECS = [
    pl.BlockSpec((NB, H), lambda i: (i, 0)),
    pl.BlockSpec((NB, 1), lambda i: (i, 0)),
    pl.BlockSpec((NB, 1), lambda i: (i, 0)),
    pl.BlockSpec((1, H), lambda i: (0, 0)),
]
_PRE_OUT_SHAPE = [
    jax.ShapeDtypeStruct((N, H), jnp.float32),
    jax.ShapeDtypeStruct((N, 1), jnp.float32),
    jax.ShapeDtypeStruct((N, 1), jnp.float32),
    jax.ShapeDtypeStruct((1, H), jnp.float32),
]
_NSPEC = pl.BlockSpec((NB, H), lambda i: (i, 0))
_WSPEC = pl.BlockSpec((H, H), lambda i: (0, 0))
_BSPEC = pl.BlockSpec((1, H), lambda i: (0, 0))


def _pre0_body(x_ref, wx_ref, bx_ref, w_ref, al_ref, ar_ref,
               h_out_ref, z_ref, el_ref, er_ref, c_ref, m_ref):
    _pre_body(x_ref, w_ref, al_ref, ar_ref, z_ref, el_ref, er_ref, c_ref,
              m_ref, wx_ref=wx_ref, bx_ref=bx_ref, h_out_ref=h_out_ref)


def _tc_pre0(x, wx, bx, w, al, ar):
    return pl.pallas_call(
        _pre0_body,
        grid=(GRID,),
        in_specs=[_NSPEC, _WSPEC, _BSPEC, _WSPEC, _BSPEC, _BSPEC],
        out_specs=[_NSPEC] + _PRE_OUT_SPECS,
        out_shape=[jax.ShapeDtypeStruct((N, H), jnp.float32)]
        + _PRE_OUT_SHAPE,
        scratch_shapes=[pltpu.SMEM((2,), jnp.float32)],
    )(x, wx, bx.reshape(1, H), w, al.reshape(1, H), ar.reshape(1, H))


def _postpre_body(accp_ref, den_ref, bg_ref, w_ref, al_ref, ar_ref,
                  h_out_ref, z_ref, el_ref, er_ref, c_ref, m_ref):
    _pre_body(None, w_ref, al_ref, ar_ref, z_ref, el_ref, er_ref, c_ref,
              m_ref, accp_ref=accp_ref, den_ref=den_ref, bg_ref=bg_ref,
              h_out_ref=h_out_ref)


def _tc_postpre(accp, den, bg, w, al, ar):
    return pl.pallas_call(
        _postpre_body,
        grid=(GRID,),
        in_specs=[
            pl.BlockSpec((NC, NB, H), lambda i: (0, i, 0)),
            pl.BlockSpec((NB, 1), lambda i: (i, 0)),
            _BSPEC, _WSPEC, _BSPEC, _BSPEC,
        ],
        out_specs=[_NSPEC] + _PRE_OUT_SPECS,
        out_shape=[jax.ShapeDtypeStruct((N, H), jnp.float32)]
        + _PRE_OUT_SHAPE,
        scratch_shapes=[pltpu.SMEM((2,), jnp.float32)],
    )(accp, den, bg.reshape(1, H), w, al.reshape(1, H), ar.reshape(1, H))


def _denred_body(denp_ref, den_ref):
    d = jnp.sum(denp_ref[...], axis=0)
    den_ref[...] = d.reshape(N, 1)


def _tc_denred(denp):
    return pl.pallas_call(
        _denred_body,
        grid=(1,),
        in_specs=[pl.BlockSpec((NW, N), lambda i: (0, 0))],
        out_specs=pl.BlockSpec((N, 1), lambda i: (0, 0)),
        out_shape=jax.ShapeDtypeStruct((N, 1), jnp.float32),
    )(denp)


def _post_body(acc_ref, den_ref, b_ref, h_ref):
    acc = acc_ref[0] + acc_ref[1]
    h_ref[...] = jnp.tanh(acc / (den_ref[...] + 1e-16) + b_ref[...])


def _tc_post(accp, den, b):
    return pl.pallas_call(
        _post_body,
        grid=(GRID,),
        in_specs=[
            pl.BlockSpec((NC, NB, H), lambda i: (0, i, 0)),
            pl.BlockSpec((NB, 1), lambda i: (i, 0)),
            pl.BlockSpec((1, H), lambda i: (0, 0)),
        ],
        out_specs=pl.BlockSpec((NB, H), lambda i: (i, 0)),
        out_shape=jax.ShapeDtypeStruct((N, H), jnp.float32),
    )(accp, den, b.reshape(1, H))


def _lstm_body(h0_ref, h1_ref, h2_ref, h3_ref, wi_ref, wf_ref, wo_ref,
               wc_ref, bi_ref, bf_ref, bo_ref, bc_ref, wout_ref, bout_ref,
               o_ref):
    mu = h0_ref[...]
    c = jnp.zeros_like(mu)
    colls = (h1_ref, h2_ref, h3_ref)
    for i in range(DEPTH):
        a = colls[i][...]
        m = mu

        def gate(w_ref, b_ref):
            y = jnp.dot(a, w_ref[i, 0:H, :],
                        preferred_element_type=jnp.float32)
            y = y + jnp.dot(m, w_ref[i, H:2 * H, :],
                            preferred_element_type=jnp.float32)
            return y + b_ref[i]

        ig = jax.nn.sigmoid(gate(wi_ref, bi_ref))
        fg = jax.nn.sigmoid(gate(wf_ref, bf_ref))
        og = jax.nn.sigmoid(gate(wo_ref, bo_ref))
        ct = jnp.tanh(gate(wc_ref, bc_ref))
        c = fg * c + ig * ct
        mu = og * jnp.tanh(c)
    out = jnp.dot(mu, wout_ref[...], preferred_element_type=jnp.float32)
    o_ref[...] = jnp.maximum(out + bout_ref[...], 0.0)


def _tc_lstm_all(h0, h1, h2, h3, wi, wf, wo, wc, bi, bf, bo, bc,
                 wout, bout):
    wspec = pl.BlockSpec((DEPTH, 2 * H, H), lambda i: (0, 0, 0))
    bspec = pl.BlockSpec((DEPTH, 1, H), lambda i: (0, 0, 0))
    return pl.pallas_call(
        _lstm_body,
        grid=(GRID,),
        in_specs=[_NSPEC, _NSPEC, _NSPEC, _NSPEC,
                  wspec, wspec, wspec, wspec,
                  bspec, bspec, bspec, bspec, _WSPEC, _BSPEC],
        out_specs=_NSPEC,
        out_shape=jax.ShapeDtypeStruct((N, H), jnp.float32),
    )(h0, h1, h2, h3, wi, wf, wo, wc,
      bi.reshape(DEPTH, 1, H), bf.reshape(DEPTH, 1, H),
      bo.reshape(DEPTH, 1, H), bc.reshape(DEPTH, 1, H),
      wout, bout.reshape(1, H))


def _sc_edge(z, el, er, src, dst, cshift):
    mesh = plsc.VectorSubcoreMesh(core_axis_name="c", subcore_axis_name="s")
    cp = pltpu.CompilerParams()
    if "needs_layout_passes" in pltpu.CompilerParams.__dataclass_fields__:
        cp = dataclasses.replace(cp, needs_layout_passes=False)

    @functools.partial(
        pl.kernel,
        compiler_params=cp,
        out_type=(
            jax.ShapeDtypeStruct((NC, N, H), jnp.float32),
            jax.ShapeDtypeStruct((NW, N), jnp.float32),
        ),
        mesh=mesh,
        scratch_types=[
            pltpu.VMEM((N,), jnp.float32),
            pltpu.VMEM((2, K), jnp.int32),
            pltpu.VMEM((2, K), jnp.int32),
            pltpu.VMEM((2, K), jnp.float32),
            pltpu.VMEM((2, K), jnp.float32),
            pltpu.VMEM((2, K, H), jnp.float32),
            pltpu.VMEM((K,), jnp.float32),
            pltpu.VMEM((16,), jnp.float32),
            pltpu.VMEM_SHARED((N, H), jnp.float32),
            pltpu.SemaphoreType.DMA,
            pltpu.SemaphoreType.DMA,
            pltpu.SemaphoreType.DMA,
            pltpu.SemaphoreType.DMA,
        ],
    )
    def k(z_hbm, el_hbm, er_hbm, src_hbm, dst_hbm, c_hbm, accout, denout,
          den_v, src_v, dst_v, el_v, er_v, rows_v, w_v, c_v,
          acc_sh, semi0, semi1, semg0, semg1):
        core = lax.axis_index("c")
        sub = lax.axis_index("s")
        wid = core * NS + sub
        semi = (semi0, semi1)
        semg = (semg0, semg1)

        pltpu.sync_copy(c_hbm, c_v)
        cvec = c_v[...]
        zero16 = jnp.zeros((16,), jnp.float32)

        @pl.loop(0, ZR)
        def _(i):
            for cc in range(H // 16):
                rows_v[0, i, pl.ds(cc * 16, 16)] = zero16

        @pl.loop(0, N, step=16)
        def _(i):
            den_v[pl.ds(i, 16)] = zero16

        for t in range((NCHUNK + NS - 1) // NS):
            cid = sub + NS * t

            @pl.when(cid < NCHUNK)
            def _():
                sl = pl.ds(cid * ZR, ZR)
                pltpu.sync_copy(rows_v.at[0, pl.ds(0, ZR)], acc_sh.at[sl])
        plsc.subcore_barrier()

        def idx_copies(g, b):
            cid = g * NW + wid
            base = cid * K
            return (
                pltpu.make_async_copy(src_hbm.at[pl.ds(base, K)],
                                      src_v.at[b], semi[b]),
                pltpu.make_async_copy(dst_hbm.at[pl.ds(base, K)],
                                      dst_v.at[b], semi[b]),
            )

        def gather_copies(b):
            return (
                pltpu.make_async_copy(el_hbm.at[src_v.at[b]], el_v.at[b],
                                      semg[b]),
                pltpu.make_async_copy(er_hbm.at[dst_v.at[b]], er_v.at[b],
                                      semg[b]),
                pltpu.make_async_copy(z_hbm.at[src_v.at[b]], rows_v.at[b],
                                      semg[b]),
            )

        def active(g):
            return g * NW + wid < NG

        def issue_idx(g, b):
            @pl.when(active(g))
            def _():
                for d in idx_copies(g, b):
                    d.start()

        def wait_idx(g, b):
            @pl.when(active(g))
            def _():
                for d in idx_copies(g, b):
                    d.wait()

        def issue_gathers(g, b):
            @pl.when(active(g))
            def _():
                for d in gather_copies(b):
                    d.start()

        def wait_gathers(g, b):
            @pl.when(active(g))
            def _():
                for d in gather_copies(b):
                    d.wait()

        def compute_scatter(g, b):
            @pl.when(active(g))
            def _():
                for j in range(K // 16):
                    sl = pl.ds(j * 16, 16)
                    didx = dst_v[b, sl]
                    t = el_v[b, sl] + er_v[b, sl]
                    e = jnp.where(t >= 0.0, t, 0.2 * t)
                    w = jnp.exp(e - cvec)
                    w_v[sl] = w
                    plsc.addupdate_scatter(den_v, [didx], w)

                @plsc.parallel_loop(0, K, unroll=2)
                def _(r):
                    wsplat = plsc.load_gather(w_v, [lax.broadcast(r, (16,))])
                    for cc in range(H // 16):
                        sl = pl.ds(cc * 16, 16)
                        rows_v[b, r, sl] = rows_v[b, r, sl] * wsplat

                pltpu.sync_copy(rows_v.at[b], acc_sh.at[dst_v.at[b]],
                                add=True)

        issue_idx(0, 0)
        issue_idx(1, 1)
        wait_idx(0, 0)
        issue_gathers(0, 0)

        @pl.loop(0, (GPW + 1) // 2)
        def _(h):
            for par in range(2):
                g = 2 * h + par
                b = par
                nb = 1 - par
                wait_idx(g + 1, nb)
                issue_gathers(g + 1, nb)
                wait_gathers(g, b)
                compute_scatter(g, b)
                issue_idx(g + 2, b)

        plsc.subcore_barrier()
        for t in range((NCHUNK + NS - 1) // NS):
            cid = sub + NS * t

            @pl.when(cid < NCHUNK)
            def _():
                rows = pl.ds(cid * ZR, ZR)
                pltpu.sync_copy(acc_sh.at[rows], accout.at[core, rows])
        pltpu.sync_copy(den_v, denout.at[wid])

    return k(z, el, er, src, dst, cshift)


def kernel(x, edge_index, Wx, bx, Wgat, al, ar, bgat, Wi, bi, Wf, bf,
           Wo, bo, Wc, bc, Wout, bout):
    src = edge_index[0]
    dst = edge_index[1]

    collector = []
    for i in range(DEPTH):
        if i == 0:
            h0, z, el, er, cmat = _tc_pre0(x, Wx, bx, Wgat[0], al[0], ar[0])
        else:
            h, z, el, er, cmat = _tc_postpre(accp, den, bgat[i - 1],
                                             Wgat[i], al[i], ar[i])
            collector.append(h)
        accp, denp = _sc_edge(z, el.reshape(N), er.reshape(N), src, dst,
                              cmat[0, :16])
        den = _tc_denred(denp)
    collector.append(_tc_post(accp, den, bgat[DEPTH - 1]))

    return _tc_lstm_all(h0, collector[0], collector[1], collector[2],
                        Wi, Wf, Wo, Wc, bi, bf, bo, bc, Wout, bout)


# speedup vs baseline: 46.1814x; 1.1222x over previous
import dataclasses
import functools

import jax
import jax.numpy as jnp
from jax import lax
from jax.experimental import pallas as pl
from jax.experimental.pallas import tpu as pltpu
from jax.experimental.pallas import tpu_sc as plsc

N = 10000
E = 320000
H = 128
DEPTH = 3

NB = 400
GRID = N // NB

NC = 2
NS = 16
NW = NC * NS
K = 128
NG = E // K
GPW = (NG + NW - 1) // NW
ZR = 80
NCHUNK = N // ZR


def _affine_body(x_ref, w_ref, b_ref, o_ref, *, act):
    y = jnp.dot(x_ref[...], w_ref[...], preferred_element_type=jnp.float32)
    y = y + b_ref[...]
    if act == "relu":
        y = jnp.maximum(y, 0.0)
    o_ref[...] = y


def _tc_affine(x, w, b, act):
    return pl.pallas_call(
        functools.partial(_affine_body, act=act),
        grid=(GRID,),
        in_specs=[
            pl.BlockSpec((NB, H), lambda i: (i, 0)),
            pl.BlockSpec((H, H), lambda i: (0, 0)),
            pl.BlockSpec((1, H), lambda i: (0, 0)),
        ],
        out_specs=pl.BlockSpec((NB, H), lambda i: (i, 0)),
        out_shape=jax.ShapeDtypeStruct((N, H), jnp.float32),
    )(x, w, b.reshape(1, H))


def _pre_body(h_ref, w_ref, al_ref, ar_ref, z_ref, el_ref, er_ref, c_ref,
              m_ref, *, wx_ref=None, bx_ref=None, h_out_ref=None,
              accp_ref=None, den_ref=None, bg_ref=None):
    i = pl.program_id(0)
    if wx_ref is not None:
        h = jnp.dot(h_ref[...], wx_ref[...],
                    preferred_element_type=jnp.float32) + bx_ref[...]
        h_out_ref[...] = h
    elif accp_ref is not None:
        acc = accp_ref[0] + accp_ref[1]
        h = jnp.tanh(acc / (den_ref[...] + 1e-16) + bg_ref[...])
        h_out_ref[...] = h
    else:
        h = h_ref[...]
    z = jnp.dot(h, w_ref[...], preferred_element_type=jnp.float32)
    z_ref[...] = z
    el = jnp.sum(z * al_ref[...], axis=1, keepdims=True)
    er = jnp.sum(z * ar_ref[...], axis=1, keepdims=True)
    el_ref[...] = el
    er_ref[...] = er
    bl = jnp.max(el)
    br = jnp.max(er)

    @pl.when(i == 0)
    def _():
        m_ref[0] = bl
        m_ref[1] = br

    @pl.when(i > 0)
    def _():
        m_ref[0] = jnp.maximum(m_ref[0], bl)
        m_ref[1] = jnp.maximum(m_ref[1], br)

    @pl.when(i == GRID - 1)
    def _():
        c_ref[...] = jnp.full((1, H), jnp.maximum(m_ref[0] + m_ref[1], 0.0),
                              dtype=jnp.float32)


_PRE_OUT_SPECS = [
    pl.BlockSpec((NB, H), lambda i: (i, 0)),
    pl.BlockSpec((NB, 1), lambda i: (i, 0)),
    pl.BlockSpec((NB, 1), lambda i: (i, 0)),
    pl.BlockSpec((1, H), lambda i: (0, 0)),
]
_PRE_OUT_SHAPE = [
    jax.ShapeDtypeStruct((N, H), jnp.float32),
    jax.ShapeDtypeStruct((N, 1), jnp.float32),
    jax.ShapeDtypeStruct((N, 1), jnp.float32),
    jax.ShapeDtypeStruct((1, H), jnp.float32),
]
_NSPEC = pl.BlockSpec((NB, H), lambda i: (i, 0))
_WSPEC = pl.BlockSpec((H, H), lambda i: (0, 0))
_BSPEC = pl.BlockSpec((1, H), lambda i: (0, 0))


def _pre0_body(x_ref, wx_ref, bx_ref, w_ref, al_ref, ar_ref,
               h_out_ref, z_ref, el_ref, er_ref, c_ref, m_ref):
    _pre_body(x_ref, w_ref, al_ref, ar_ref, z_ref, el_ref, er_ref, c_ref,
              m_ref, wx_ref=wx_ref, bx_ref=bx_ref, h_out_ref=h_out_ref)


def _tc_pre0(x, wx, bx, w, al, ar):
    return pl.pallas_call(
        _pre0_body,
        grid=(GRID,),
        in_specs=[_NSPEC, _WSPEC, _BSPEC, _WSPEC, _BSPEC, _BSPEC],
        out_specs=[_NSPEC] + _PRE_OUT_SPECS,
        out_shape=[jax.ShapeDtypeStruct((N, H), jnp.float32)]
        + _PRE_OUT_SHAPE,
        scratch_shapes=[pltpu.SMEM((2,), jnp.float32)],
    )(x, wx, bx.reshape(1, H), w, al.reshape(1, H), ar.reshape(1, H))


def _postpre_body(accp_ref, den_ref, bg_ref, w_ref, al_ref, ar_ref,
                  h_out_ref, z_ref, el_ref, er_ref, c_ref, m_ref):
    _pre_body(None, w_ref, al_ref, ar_ref, z_ref, el_ref, er_ref, c_ref,
              m_ref, accp_ref=accp_ref, den_ref=den_ref, bg_ref=bg_ref,
              h_out_ref=h_out_ref)


def _tc_postpre(accp, den, bg, w, al, ar):
    return pl.pallas_call(
        _postpre_body,
        grid=(GRID,),
        in_specs=[
            pl.BlockSpec((NC, NB, H), lambda i: (0, i, 0)),
            _DENSPEC,
            _BSPEC, _WSPEC, _BSPEC, _BSPEC,
        ],
        out_specs=[_NSPEC] + _PRE_OUT_SPECS,
        out_shape=[jax.ShapeDtypeStruct((N, H), jnp.float32)]
        + _PRE_OUT_SHAPE,
        scratch_shapes=[pltpu.SMEM((2,), jnp.float32)],
    )(accp, den, bg.reshape(1, H), w, al.reshape(1, H), ar.reshape(1, H))


def _denred_body(denp_ref, den_ref):
    d = jnp.sum(denp_ref[...], axis=0)
    den_ref[...] = d.reshape(N, 1)


def _tc_denred(denp):
    return pl.pallas_call(
        _denred_body,
        grid=(1,),
        in_specs=[pl.BlockSpec((NW, N), lambda i: (0, 0))],
        out_specs=pl.BlockSpec((N, 1), lambda i: (0, 0)),
        out_shape=jax.ShapeDtypeStruct((N, 1), jnp.float32),
    )(denp)


_DENSPEC = pl.BlockSpec((NB, 1), lambda i: (i, 0))


def _post_body(acc_ref, den_ref, b_ref, h_ref):
    acc = acc_ref[0] + acc_ref[1]
    h_ref[...] = jnp.tanh(acc / (den_ref[...] + 1e-16) + b_ref[...])


def _tc_post(accp, den, b):
    return pl.pallas_call(
        _post_body,
        grid=(GRID,),
        in_specs=[
            pl.BlockSpec((NC, NB, H), lambda i: (0, i, 0)),
            _DENSPEC,
            pl.BlockSpec((1, H), lambda i: (0, 0)),
        ],
        out_specs=pl.BlockSpec((NB, H), lambda i: (i, 0)),
        out_shape=jax.ShapeDtypeStruct((N, H), jnp.float32),
    )(accp, den, b.reshape(1, H))


def _lstm_body(h0_ref, h1_ref, h2_ref, h3_ref, wi_ref, wf_ref, wo_ref,
               wc_ref, bi_ref, bf_ref, bo_ref, bc_ref, wout_ref, bout_ref,
               o_ref):
    mu = h0_ref[...]
    c = jnp.zeros_like(mu)
    colls = (h1_ref, h2_ref, h3_ref)
    for i in range(DEPTH):
        a = colls[i][...]
        m = mu

        def gate(w_ref, b_ref):
            y = jnp.dot(a, w_ref[i, 0:H, :],
                        preferred_element_type=jnp.float32)
            y = y + jnp.dot(m, w_ref[i, H:2 * H, :],
                            preferred_element_type=jnp.float32)
            return y + b_ref[i]

        ig = jax.nn.sigmoid(gate(wi_ref, bi_ref))
        fg = jax.nn.sigmoid(gate(wf_ref, bf_ref))
        og = jax.nn.sigmoid(gate(wo_ref, bo_ref))
        ct = jnp.tanh(gate(wc_ref, bc_ref))
        c = fg * c + ig * ct
        mu = og * jnp.tanh(c)
    out = jnp.dot(mu, wout_ref[...], preferred_element_type=jnp.float32)
    o_ref[...] = jnp.maximum(out + bout_ref[...], 0.0)


def _tc_lstm_all(h0, h1, h2, h3, wi, wf, wo, wc, bi, bf, bo, bc,
                 wout, bout):
    wspec = pl.BlockSpec((DEPTH, 2 * H, H), lambda i: (0, 0, 0))
    bspec = pl.BlockSpec((DEPTH, 1, H), lambda i: (0, 0, 0))
    return pl.pallas_call(
        _lstm_body,
        grid=(GRID,),
        in_specs=[_NSPEC, _NSPEC, _NSPEC, _NSPEC,
                  wspec, wspec, wspec, wspec,
                  bspec, bspec, bspec, bspec, _WSPEC, _BSPEC],
        out_specs=_NSPEC,
        out_shape=jax.ShapeDtypeStruct((N, H), jnp.float32),
    )(h0, h1, h2, h3, wi, wf, wo, wc,
      bi.reshape(DEPTH, 1, H), bf.reshape(DEPTH, 1, H),
      bo.reshape(DEPTH, 1, H), bc.reshape(DEPTH, 1, H),
      wout, bout.reshape(1, H))


def _sc_edge(z, el, er, src, dst, cshift):
    mesh = plsc.VectorSubcoreMesh(core_axis_name="c", subcore_axis_name="s")
    cp = pltpu.CompilerParams()
    if "needs_layout_passes" in pltpu.CompilerParams.__dataclass_fields__:
        cp = dataclasses.replace(cp, needs_layout_passes=False)

    @functools.partial(
        pl.kernel,
        compiler_params=cp,
        out_type=(
            jax.ShapeDtypeStruct((NC, N, H), jnp.float32),
            jax.ShapeDtypeStruct((NW, N), jnp.float32),
        ),
        mesh=mesh,
        scratch_types=[
            pltpu.VMEM((N,), jnp.float32),
            pltpu.VMEM((2, K), jnp.int32),
            pltpu.VMEM((2, K), jnp.int32),
            pltpu.VMEM((2, K), jnp.float32),
            pltpu.VMEM((2, K), jnp.float32),
            pltpu.VMEM((2, K, H), jnp.float32),
            pltpu.VMEM((K,), jnp.float32),
            pltpu.VMEM((16,), jnp.float32),
            pltpu.VMEM((2, K), jnp.int32),
            pltpu.VMEM_SHARED((N, H), jnp.float32),
            pltpu.SemaphoreType.DMA,
            pltpu.SemaphoreType.DMA,
            pltpu.SemaphoreType.DMA,
            pltpu.SemaphoreType.DMA,
            pltpu.SemaphoreType.DMA,
            pltpu.SemaphoreType.DMA,
        ],
    )
    def k(z_hbm, el_hbm, er_hbm, src_hbm, dst_hbm, c_hbm, accout, denout,
          den_v, src_v, dst_v, el_v, er_v, rows_v, w_v, c_v, dsc_v,
          acc_sh, semi0, semi1, semg0, semg1, sems0, sems1):
        core = lax.axis_index("c")
        sub = lax.axis_index("s")
        wid = core * NS + sub
        semi = (semi0, semi1)
        semg = (semg0, semg1)
        sems = (sems0, sems1)

        pltpu.sync_copy(c_hbm, c_v)
        cvec = c_v[...]
        zero16 = jnp.zeros((16,), jnp.float32)

        @pl.loop(0, ZR)
        def _(i):
            for cc in range(H // 16):
                rows_v[0, i, pl.ds(cc * 16, 16)] = zero16

        @pl.loop(0, N, step=16)
        def _(i):
            den_v[pl.ds(i, 16)] = zero16

        for t in range((NCHUNK + NS - 1) // NS):
            cid = sub + NS * t

            @pl.when(cid < NCHUNK)
            def _():
                sl = pl.ds(cid * ZR, ZR)
                pltpu.sync_copy(rows_v.at[0, pl.ds(0, ZR)], acc_sh.at[sl])
        plsc.subcore_barrier()

        def idx_copies(g, b):
            cid = g * NW + wid
            base = cid * K
            return (
                pltpu.make_async_copy(src_hbm.at[pl.ds(base, K)],
                                      src_v.at[b], semi[b]),
                pltpu.make_async_copy(dst_hbm.at[pl.ds(base, K)],
                                      dst_v.at[b], semi[b]),
            )

        def gather_copies(b):
            return (
                pltpu.make_async_copy(el_hbm.at[src_v.at[b]], el_v.at[b],
                                      semg[b]),
                pltpu.make_async_copy(er_hbm.at[dst_v.at[b]], er_v.at[b],
                                      semg[b]),
                pltpu.make_async_copy(z_hbm.at[src_v.at[b]], rows_v.at[b],
                                      semg[b]),
            )

        def active(g):
            return g * NW + wid < NG

        def issue_idx(g, b):
            @pl.when(active(g))
            def _():
                for d in idx_copies(g, b):
                    d.start()

        def wait_idx(g, b):
            @pl.when(active(g))
            def _():
                for d in idx_copies(g, b):
                    d.wait()

        def issue_gathers(g, b):
            @pl.when(active(g))
            def _():
                for d in gather_copies(b):
                    d.start()

        def wait_gathers(g, b):
            @pl.when(active(g))
            def _():
                for d in gather_copies(b):
                    d.wait()

        def scat_copy(b):
            return pltpu.make_async_copy(rows_v.at[b],
                                         acc_sh.at[dsc_v.at[b]], sems[b])

        def wait_scat(g, b):
            @pl.when((g >= 0) & active(g))
            def _():
                scat_copy(b).wait()

        def compute_scatter(g, b):
            @pl.when(active(g))
            def _():
                for j in range(K // 16):
                    sl = pl.ds(j * 16, 16)
                    didx = dst_v[b, sl]
                    t = el_v[b, sl] + er_v[b, sl]
                    e = jnp.where(t >= 0.0, t, 0.2 * t)
                    w = jnp.exp(e - cvec)
                    w_v[sl] = w
                    plsc.addupdate_scatter(den_v, [didx], w)

                for j in range(K // 16):
                    sl = pl.ds(j * 16, 16)
                    dsc_v[b, sl] = dst_v[b, sl]

                @plsc.parallel_loop(0, K, unroll=2)
                def _(r):
                    wsplat = plsc.load_gather(w_v, [lax.broadcast(r, (16,))])
                    for cc in range(H // 16):
                        sl = pl.ds(cc * 16, 16)
                        rows_v[b, r, sl] = rows_v[b, r, sl] * wsplat

                pltpu.async_copy(rows_v.at[b], acc_sh.at[dsc_v.at[b]],
                                 sems[b], add=True)

        issue_idx(0, 0)
        issue_idx(1, 1)
        wait_idx(0, 0)
        issue_gathers(0, 0)

        @pl.loop(0, (GPW + 1) // 2)
        def _(h):
            for par in range(2):
                g = 2 * h + par
                b = par
                nb = 1 - par
                wait_idx(g + 1, nb)
                wait_scat(g - 1, nb)
                issue_gathers(g + 1, nb)
                wait_gathers(g, b)
                compute_scatter(g, b)
                issue_idx(g + 2, b)

        plsc.subcore_barrier()
        for t in range((NCHUNK + NS - 1) // NS):
            cid = sub + NS * t

            @pl.when(cid < NCHUNK)
            def _():
                rows = pl.ds(cid * ZR, ZR)
                pltpu.sync_copy(acc_sh.at[rows], accout.at[core, rows])
        pltpu.sync_copy(den_v, denout.at[wid])

    return k(z, el, er, src, dst, cshift)


def kernel(x, edge_index, Wx, bx, Wgat, al, ar, bgat, Wi, bi, Wf, bf,
           Wo, bo, Wc, bc, Wout, bout):
    src = edge_index[0]
    dst = edge_index[1]

    collector = []
    for i in range(DEPTH):
        if i == 0:
            h0, z, el, er, cmat = _tc_pre0(x, Wx, bx, Wgat[0], al[0], ar[0])
        else:
            h, z, el, er, cmat = _tc_postpre(accp, den, bgat[i - 1],
                                             Wgat[i], al[i], ar[i])
            collector.append(h)
        accp, denp = _sc_edge(z, el.reshape(N), er.reshape(N), src, dst,
                              cmat[0, :16])
        den = _tc_denred(denp)
    collector.append(_tc_post(accp, den, bgat[DEPTH - 1]))

    return _tc_lstm_all(h0, collector[0], collector[1], collector[2],
                        Wi, Wf, Wo, Wc, bi, bf, bo, bc, Wout, bout)


# speedup vs baseline: 46.2446x; 1.0014x over previous
import dataclasses
import functools

import jax
import jax.numpy as jnp
from jax import lax
from jax.experimental import pallas as pl
from jax.experimental.pallas import tpu as pltpu
from jax.experimental.pallas import tpu_sc as plsc

N = 10000
E = 320000
H = 128
DEPTH = 3

NB = 400
GRID = N // NB

NC = 2
NS = 16
NW = NC * NS
K = 128
NG = E // K
GPW = (NG + NW - 1) // NW
ZR = 80
NCHUNK = N // ZR


def _affine_body(x_ref, w_ref, b_ref, o_ref, *, act):
    y = jnp.dot(x_ref[...], w_ref[...], preferred_element_type=jnp.float32)
    y = y + b_ref[...]
    if act == "relu":
        y = jnp.maximum(y, 0.0)
    o_ref[...] = y


def _tc_affine(x, w, b, act):
    return pl.pallas_call(
        functools.partial(_affine_body, act=act),
        grid=(GRID,),
        in_specs=[
            pl.BlockSpec((NB, H), lambda i: (i, 0)),
            pl.BlockSpec((H, H), lambda i: (0, 0)),
            pl.BlockSpec((1, H), lambda i: (0, 0)),
        ],
        out_specs=pl.BlockSpec((NB, H), lambda i: (i, 0)),
        out_shape=jax.ShapeDtypeStruct((N, H), jnp.float32),
    )(x, w, b.reshape(1, H))


def _pre_body(h_ref, w_ref, al_ref, ar_ref, z_ref, el_ref, er_ref, c_ref,
              m_ref, *, wx_ref=None, bx_ref=None, h_out_ref=None,
              accp_ref=None, den_ref=None, bg_ref=None):
    i = pl.program_id(0)
    if wx_ref is not None:
        h = jnp.dot(h_ref[...], wx_ref[...],
                    preferred_element_type=jnp.float32) + bx_ref[...]
        h_out_ref[...] = h
    elif accp_ref is not None:
        acc = accp_ref[0] + accp_ref[1]
        h = jnp.tanh(acc / (den_ref[...] + 1e-16) + bg_ref[...])
        h_out_ref[...] = h
    else:
        h = h_ref[...]
    z = jnp.dot(h, w_ref[...], preferred_element_type=jnp.float32)
    z_ref[...] = z
    el = jnp.sum(z * al_ref[...], axis=1, keepdims=True)
    er = jnp.sum(z * ar_ref[...], axis=1, keepdims=True)
    el_ref[...] = el
    er_ref[...] = er
    bl = jnp.max(el)
    br = jnp.max(er)

    @pl.when(i == 0)
    def _():
        m_ref[0] = bl
        m_ref[1] = br

    @pl.when(i > 0)
    def _():
        m_ref[0] = jnp.maximum(m_ref[0], bl)
        m_ref[1] = jnp.maximum(m_ref[1], br)

    @pl.when(i == GRID - 1)
    def _():
        c_ref[...] = jnp.full((1, H), jnp.maximum(m_ref[0] + m_ref[1], 0.0),
                              dtype=jnp.float32)


_PRE_OUT_SPECS = [
    pl.BlockSpec((NB, H), lambda i: (i, 0)),
    pl.BlockSpec((NB, 1), lambda i: (i, 0)),
    pl.BlockSpec((NB, 1), lambda i: (i, 0)),
    pl.BlockSpec((1, H), lambda i: (0, 0)),
]
_PRE_OUT_SHAPE = [
    jax.ShapeDtypeStruct((N, H), jnp.float32),
    jax.ShapeDtypeStruct((N, 1), jnp.float32),
    jax.ShapeDtypeStruct((N, 1), jnp.float32),
    jax.ShapeDtypeStruct((1, H), jnp.float32),
]
_NSPEC = pl.BlockSpec((NB, H), lambda i: (i, 0))
_WSPEC = pl.BlockSpec((H, H), lambda i: (0, 0))
_BSPEC = pl.BlockSpec((1, H), lambda i: (0, 0))


def _pre0_body(x_ref, wx_ref, bx_ref, w_ref, al_ref, ar_ref,
               h_out_ref, z_ref, el_ref, er_ref, c_ref, m_ref):
    _pre_body(x_ref, w_ref, al_ref, ar_ref, z_ref, el_ref, er_ref, c_ref,
              m_ref, wx_ref=wx_ref, bx_ref=bx_ref, h_out_ref=h_out_ref)


def _tc_pre0(x, wx, bx, w, al, ar):
    return pl.pallas_call(
        _pre0_body,
        grid=(GRID,),
        in_specs=[_NSPEC, _WSPEC, _BSPEC, _WSPEC, _BSPEC, _BSPEC],
        out_specs=[_NSPEC] + _PRE_OUT_SPECS,
        out_shape=[jax.ShapeDtypeStruct((N, H), jnp.float32)]
        + _PRE_OUT_SHAPE,
        scratch_shapes=[pltpu.SMEM((2,), jnp.float32)],
    )(x, wx, bx.reshape(1, H), w, al.reshape(1, H), ar.reshape(1, H))


def _postpre_body(accp_ref, den_ref, bg_ref, w_ref, al_ref, ar_ref,
                  h_out_ref, z_ref, el_ref, er_ref, c_ref, m_ref):
    _pre_body(None, w_ref, al_ref, ar_ref, z_ref, el_ref, er_ref, c_ref,
              m_ref, accp_ref=accp_ref, den_ref=den_ref, bg_ref=bg_ref,
              h_out_ref=h_out_ref)


def _tc_postpre(accp, den, bg, w, al, ar):
    return pl.pallas_call(
        _postpre_body,
        grid=(GRID,),
        in_specs=[
            pl.BlockSpec((NC, NB, H), lambda i: (0, i, 0)),
            _DENSPEC,
            _BSPEC, _WSPEC, _BSPEC, _BSPEC,
        ],
        out_specs=[_NSPEC] + _PRE_OUT_SPECS,
        out_shape=[jax.ShapeDtypeStruct((N, H), jnp.float32)]
        + _PRE_OUT_SHAPE,
        scratch_shapes=[pltpu.SMEM((2,), jnp.float32)],
    )(accp, den, bg.reshape(1, H), w, al.reshape(1, H), ar.reshape(1, H))


def _denred_body(denp_ref, den_ref):
    d = jnp.sum(denp_ref[...], axis=0)
    den_ref[...] = d.reshape(N, 1)


def _tc_denred(denp):
    return pl.pallas_call(
        _denred_body,
        grid=(1,),
        in_specs=[pl.BlockSpec((NW, N), lambda i: (0, 0))],
        out_specs=pl.BlockSpec((N, 1), lambda i: (0, 0)),
        out_shape=jax.ShapeDtypeStruct((N, 1), jnp.float32),
    )(denp)


_DENSPEC = pl.BlockSpec((NB, 1), lambda i: (i, 0))


def _post_body(acc_ref, den_ref, b_ref, h_ref):
    acc = acc_ref[0] + acc_ref[1]
    h_ref[...] = jnp.tanh(acc / (den_ref[...] + 1e-16) + b_ref[...])


def _tc_post(accp, den, b):
    return pl.pallas_call(
        _post_body,
        grid=(GRID,),
        in_specs=[
            pl.BlockSpec((NC, NB, H), lambda i: (0, i, 0)),
            _DENSPEC,
            pl.BlockSpec((1, H), lambda i: (0, 0)),
        ],
        out_specs=pl.BlockSpec((NB, H), lambda i: (i, 0)),
        out_shape=jax.ShapeDtypeStruct((N, H), jnp.float32),
    )(accp, den, b.reshape(1, H))


def _lstm_body(h0_ref, h1_ref, h2_ref, h3_ref, wi_ref, wf_ref, wo_ref,
               wc_ref, bi_ref, bf_ref, bo_ref, bc_ref, wout_ref, bout_ref,
               o_ref):
    mu = h0_ref[...]
    c = jnp.zeros_like(mu)
    colls = (h1_ref, h2_ref, h3_ref)
    for i in range(DEPTH):
        a = colls[i][...]
        m = mu

        def gate(w_ref, b_ref):
            y = jnp.dot(a, w_ref[i, 0:H, :],
                        preferred_element_type=jnp.float32)
            y = y + jnp.dot(m, w_ref[i, H:2 * H, :],
                            preferred_element_type=jnp.float32)
            return y + b_ref[i]

        ig = jax.nn.sigmoid(gate(wi_ref, bi_ref))
        fg = jax.nn.sigmoid(gate(wf_ref, bf_ref))
        og = jax.nn.sigmoid(gate(wo_ref, bo_ref))
        ct = jnp.tanh(gate(wc_ref, bc_ref))
        c = fg * c + ig * ct
        mu = og * jnp.tanh(c)
    out = jnp.dot(mu, wout_ref[...], preferred_element_type=jnp.float32)
    o_ref[...] = jnp.maximum(out + bout_ref[...], 0.0)


def _tc_lstm_all(h0, h1, h2, h3, wi, wf, wo, wc, bi, bf, bo, bc,
                 wout, bout):
    wspec = pl.BlockSpec((DEPTH, 2 * H, H), lambda i: (0, 0, 0))
    bspec = pl.BlockSpec((DEPTH, 1, H), lambda i: (0, 0, 0))
    return pl.pallas_call(
        _lstm_body,
        grid=(GRID,),
        in_specs=[_NSPEC, _NSPEC, _NSPEC, _NSPEC,
                  wspec, wspec, wspec, wspec,
                  bspec, bspec, bspec, bspec, _WSPEC, _BSPEC],
        out_specs=_NSPEC,
        out_shape=jax.ShapeDtypeStruct((N, H), jnp.float32),
    )(h0, h1, h2, h3, wi, wf, wo, wc,
      bi.reshape(DEPTH, 1, H), bf.reshape(DEPTH, 1, H),
      bo.reshape(DEPTH, 1, H), bc.reshape(DEPTH, 1, H),
      wout, bout.reshape(1, H))


def _sc_edge(z, el, er, src, dst, cshift):
    mesh = plsc.VectorSubcoreMesh(core_axis_name="c", subcore_axis_name="s")
    cp = pltpu.CompilerParams()
    if "needs_layout_passes" in pltpu.CompilerParams.__dataclass_fields__:
        cp = dataclasses.replace(cp, needs_layout_passes=False)

    @functools.partial(
        pl.kernel,
        compiler_params=cp,
        out_type=(
            jax.ShapeDtypeStruct((NC, N, H), jnp.float32),
            jax.ShapeDtypeStruct((NW, N), jnp.float32),
        ),
        mesh=mesh,
        scratch_types=[
            pltpu.VMEM((N,), jnp.float32),
            pltpu.VMEM((2, K), jnp.int32),
            pltpu.VMEM((2, K), jnp.int32),
            pltpu.VMEM((2, K), jnp.float32),
            pltpu.VMEM((2, K), jnp.float32),
            pltpu.VMEM((2, K, H), jnp.float32),
            pltpu.VMEM((K,), jnp.float32),
            pltpu.VMEM((16,), jnp.float32),
            pltpu.VMEM((2, K), jnp.int32),
            pltpu.VMEM_SHARED((N, H), jnp.float32),
            pltpu.SemaphoreType.DMA,
            pltpu.SemaphoreType.DMA,
            pltpu.SemaphoreType.DMA,
            pltpu.SemaphoreType.DMA,
            pltpu.SemaphoreType.DMA,
            pltpu.SemaphoreType.DMA,
        ],
    )
    def k(z_hbm, el_hbm, er_hbm, src_hbm, dst_hbm, c_hbm, accout, denout,
          den_v, src_v, dst_v, el_v, er_v, rows_v, w_v, c_v, dsc_v,
          acc_sh, semi0, semi1, semg0, semg1, sems0, sems1):
        core = lax.axis_index("c")
        sub = lax.axis_index("s")
        wid = core * NS + sub
        semi = (semi0, semi1)
        semg = (semg0, semg1)
        sems = (sems0, sems1)

        pltpu.sync_copy(c_hbm, c_v)
        cvec = c_v[...]
        zero16 = jnp.zeros((16,), jnp.float32)

        @pl.loop(0, ZR)
        def _(i):
            for cc in range(H // 16):
                rows_v[0, i, pl.ds(cc * 16, 16)] = zero16

        @pl.loop(0, N, step=16)
        def _(i):
            den_v[pl.ds(i, 16)] = zero16

        for t in range((NCHUNK + NS - 1) // NS):
            cid = sub + NS * t

            @pl.when(cid < NCHUNK)
            def _():
                sl = pl.ds(cid * ZR, ZR)
                pltpu.sync_copy(rows_v.at[0, pl.ds(0, ZR)], acc_sh.at[sl])
        plsc.subcore_barrier()

        def idx_copies(g, b):
            cid = g * NW + wid
            base = cid * K
            return (
                pltpu.make_async_copy(src_hbm.at[pl.ds(base, K)],
                                      src_v.at[b], semi[b]),
                pltpu.make_async_copy(dst_hbm.at[pl.ds(base, K)],
                                      dst_v.at[b], semi[b]),
            )

        def gather_copies(b):
            return (
                pltpu.make_async_copy(el_hbm.at[src_v.at[b]], el_v.at[b],
                                      semg[b]),
                pltpu.make_async_copy(er_hbm.at[dst_v.at[b]], er_v.at[b],
                                      semg[b]),
                pltpu.make_async_copy(z_hbm.at[src_v.at[b]], rows_v.at[b],
                                      semg[b]),
            )

        def active(g):
            return g * NW + wid < NG

        def issue_idx(g, b):
            @pl.when(active(g))
            def _():
                for d in idx_copies(g, b):
                    d.start()

        def wait_idx(g, b):
            @pl.when(active(g))
            def _():
                for d in idx_copies(g, b):
                    d.wait()

        def issue_gathers(g, b):
            @pl.when(active(g))
            def _():
                for d in gather_copies(b):
                    d.start()

        def wait_gathers(g, b):
            @pl.when(active(g))
            def _():
                for d in gather_copies(b):
                    d.wait()

        def scat_copy(b):
            return pltpu.make_async_copy(rows_v.at[b],
                                         acc_sh.at[dsc_v.at[b]], sems[b])

        def wait_scat(g, b):
            @pl.when((g >= 0) & active(g))
            def _():
                scat_copy(b).wait()

        def compute_scatter(g, b):
            @pl.when(active(g))
            def _():
                for j in range(K // 16):
                    sl = pl.ds(j * 16, 16)
                    didx = dst_v[b, sl]
                    t = el_v[b, sl] + er_v[b, sl]
                    e = jnp.where(t >= 0.0, t, 0.2 * t)
                    w = jnp.exp(e - cvec)
                    w_v[sl] = w
                    plsc.addupdate_scatter(den_v, [didx], w)

                for j in range(K // 16):
                    sl = pl.ds(j * 16, 16)
                    dsc_v[b, sl] = dst_v[b, sl]

                @plsc.parallel_loop(0, K, unroll=4)
                def _(r):
                    wsplat = plsc.load_gather(w_v, [lax.broadcast(r, (16,))])
                    for cc in range(H // 16):
                        sl = pl.ds(cc * 16, 16)
                        rows_v[b, r, sl] = rows_v[b, r, sl] * wsplat

                pltpu.async_copy(rows_v.at[b], acc_sh.at[dsc_v.at[b]],
                                 sems[b], add=True)

        issue_idx(0, 0)
        issue_idx(1, 1)
        wait_idx(0, 0)
        issue_gathers(0, 0)

        @pl.loop(0, (GPW + 1) // 2)
        def _(h):
            for par in range(2):
                g = 2 * h + par
                b = par
                nb = 1 - par
                wait_idx(g + 1, nb)
                wait_scat(g - 1, nb)
                issue_gathers(g + 1, nb)
                wait_gathers(g, b)
                compute_scatter(g, b)
                issue_idx(g + 2, b)

        plsc.subcore_barrier()
        for t in range((NCHUNK + NS - 1) // NS):
            cid = sub + NS * t

            @pl.when(cid < NCHUNK)
            def _():
                rows = pl.ds(cid * ZR, ZR)
                pltpu.sync_copy(acc_sh.at[rows], accout.at[core, rows])
        pltpu.sync_copy(den_v, denout.at[wid])

    return k(z, el, er, src, dst, cshift)


def kernel(x, edge_index, Wx, bx, Wgat, al, ar, bgat, Wi, bi, Wf, bf,
           Wo, bo, Wc, bc, Wout, bout):
    src = edge_index[0]
    dst = edge_index[1]

    collector = []
    for i in range(DEPTH):
        if i == 0:
            h0, z, el, er, cmat = _tc_pre0(x, Wx, bx, Wgat[0], al[0], ar[0])
        else:
            h, z, el, er, cmat = _tc_postpre(accp, den, bgat[i - 1],
                                             Wgat[i], al[i], ar[i])
            collector.append(h)
        accp, denp = _sc_edge(z, el.reshape(N), er.reshape(N), src, dst,
                              cmat[0, :16])
        den = _tc_denred(denp)
    collector.append(_tc_post(accp, den, bgat[DEPTH - 1]))

    return _tc_lstm_all(h0, collector[0], collector[1], collector[2],
                        Wi, Wf, Wo, Wc, bi, bf, bo, bc, Wout, bout)
